# baseline (speedup 1.0000x reference)
import functools

import jax
import jax.numpy as jnp
from jax import lax
from jax.experimental import pallas as pl
from jax.experimental.pallas import tpu as pltpu
from jax.experimental.pallas import tpu_sc as plsc

_NC = 2
_NS = 16
_L = 16
_GB = 8
_NG = 2
_TW = 128


@functools.partial(jax.jit, static_argnames=("b_dim", "f_dim", "d"))
def _gather_rows(xt, table, b_dim, f_dim, d):
    nw = _NC * _NS
    b_per_w = b_dim // nw
    per_w = b_per_w * f_dim
    n_groups = b_per_w // _GB
    cg = _GB * f_dim
    gi = cg // _NG

    mesh = plsc.VectorSubcoreMesh(
        core_axis_name="c", subcore_axis_name="s",
        num_cores=_NC, num_subcores=_NS,
    )

    @functools.partial(
        pl.kernel,
        mesh=mesh,
        out_type=jax.ShapeDtypeStruct((b_dim, f_dim, d), jnp.float32),
        scratch_types=[
            pltpu.VMEM((f_dim, b_per_w), jnp.float32),
            pltpu.VMEM((per_w + _L,), jnp.int32),
            pltpu.VMEM((2, cg, d), jnp.float32),
            pltpu.SemaphoreType.DMA,
            pltpu.SemaphoreType.DMA,
        ],
        compiler_params=pltpu.CompilerParams(use_tc_tiling_on_sc=False,
                                             needs_layout_passes=False),
    )
    def emb_kernel(xt_hbm, table_hbm, out_hbm, fidx_v, idx_v, rows_v,
                   gsem, osem):
        wid = lax.axis_index("s") * _NC + lax.axis_index("c")
        b0 = wid * b_per_w

        pltpu.sync_copy(xt_hbm.at[:, pl.ds(b0, b_per_w)], fidx_v)

        lane = lax.iota(jnp.int32, _L)
        f_mask = lane < (f_dim - _L)

        def reorder(bl, carry):
            bvec = lane * 0 + bl
            v0 = plsc.load_gather(fidx_v, [lane, bvec])
            v1 = plsc.load_gather(fidx_v, [lane + _L, bvec], mask=f_mask)
            idx_v[pl.ds(bl * f_dim, _L)] = plsc.bitcast(v0, jnp.int32)
            idx_v[pl.ds(bl * f_dim + _L, _L)] = plsc.bitcast(v1, jnp.int32)
            return carry

        lax.fori_loop(0, b_per_w, reorder, 0)

        def fire(g, slot):
            for j in range(_NG):
                pltpu.async_copy(
                    table_hbm.at[idx_v.at[pl.ds(g * cg + j * gi, gi)]],
                    rows_v.at[slot, pl.ds(j * gi, gi)],
                    gsem,
                )

        def wait_gathers(slot):
            for j in range(_NG):
                pltpu.make_async_copy(
                    table_hbm.at[idx_v.at[pl.ds(j * gi, gi)]],
                    rows_v.at[slot, pl.ds(j * gi, gi)],
                    gsem,
                ).wait()

        def start_wb(g, slot):
            for k in range(_GB):
                pltpu.async_copy(
                    rows_v.at[slot, pl.ds(k * f_dim, f_dim)],
                    out_hbm.at[b0 + g * _GB + k],
                    osem,
                )

        def wait_wb(slot):
            for k in range(_GB):
                pltpu.make_async_copy(
                    rows_v.at[slot, pl.ds(k * f_dim, f_dim)],
                    out_hbm.at[b0],
                    osem,
                ).wait()

        fire(0, 0)
        fire(1, 1)
        wait_gathers(0)
        start_wb(0, 0)

        def body(i, carry):
            gb = 1 + 2 * i
            for b in range(2):
                g = gb + b
                slot = (1 + b) % 2
                other = 1 - slot
                wait_wb(other)
                fire(g + 1, other)
                wait_gathers(slot)
                start_wb(g, slot)
            return carry

        lax.fori_loop(0, (n_groups - 2) // 2, body, 0)

        g_last = n_groups - 1
        slot = g_last % 2
        wait_gathers(slot)
        start_wb(g_last, slot)
        wait_wb(1 - slot)
        wait_wb(slot)

    return emb_kernel(xt, table)


def kernel(x, embedding):
    b, f = x.shape
    v, d = embedding.shape
    xt = lax.bitcast_convert_type(x.astype(jnp.int32), jnp.float32).T
    cut = 499712
    h0 = lax.optimization_barrier(embedding[:cut])
    h1 = lax.optimization_barrier(embedding[cut:])
    table2 = jnp.concatenate([h0, h1], axis=0)
    return _gather_rows(xt, table2, b, f, d)


# speedup vs baseline: 1.4735x; 1.4735x over previous
import functools

import jax
import jax.numpy as jnp
from jax import lax
from jax.experimental import pallas as pl
from jax.experimental.pallas import tpu as pltpu
from jax.experimental.pallas import tpu_sc as plsc

_NC = 2
_NS = 16
_L = 16
_GB = 16
_NG = 4


@functools.partial(jax.jit, static_argnames=("b_dim", "f_dim", "d"))
def _gather_rows(xt, table, b_dim, f_dim, d):
    nw = _NC * _NS
    b_per_w = b_dim // nw
    per_w = b_per_w * f_dim
    n_groups = b_per_w // _GB
    cg = _GB * f_dim
    gi = cg // _NG

    mesh = plsc.VectorSubcoreMesh(
        core_axis_name="c", subcore_axis_name="s",
        num_cores=_NC, num_subcores=_NS,
    )

    @functools.partial(
        pl.kernel,
        mesh=mesh,
        out_type=jax.ShapeDtypeStruct((b_dim, f_dim, d), jnp.float32),
        scratch_types=[
            pltpu.VMEM((f_dim, b_per_w), jnp.float32),
            pltpu.VMEM((per_w + _L,), jnp.int32),
            pltpu.VMEM((2, _GB * f_dim, d), jnp.float32),
            pltpu.SemaphoreType.DMA,
            pltpu.SemaphoreType.DMA,
        ],
        compiler_params=pltpu.CompilerParams(use_tc_tiling_on_sc=False,
                                             needs_layout_passes=False),
    )
    def emb_kernel(xt_hbm, table_hbm, out_hbm, fidx_v, idx_v, rows_v,
                   gsem, osem):
        wid = lax.axis_index("s") * _NC + lax.axis_index("c")
        b0 = wid * b_per_w

        pltpu.sync_copy(xt_hbm.at[:, pl.ds(b0, b_per_w)], fidx_v)

        lane = lax.iota(jnp.int32, _L)
        f_mask = lane < (f_dim - _L)

        def reorder(bl, carry):
            bvec = lane * 0 + bl
            v0 = plsc.load_gather(fidx_v, [lane, bvec])
            v1 = plsc.load_gather(fidx_v, [lane + _L, bvec], mask=f_mask)
            idx_v[pl.ds(bl * f_dim, _L)] = plsc.bitcast(v0, jnp.int32)
            idx_v[pl.ds(bl * f_dim + _L, _L)] = plsc.bitcast(v1, jnp.int32)
            return carry

        lax.fori_loop(0, b_per_w, reorder, 0)

        def fire(g, slot):
            for j in range(_NG):
                pltpu.async_copy(
                    table_hbm.at[idx_v.at[pl.ds(g * cg + j * gi, gi)]],
                    rows_v.at[slot, pl.ds(j * gi, gi)],
                    gsem,
                )

        def wait_gathers(slot):
            for j in range(_NG):
                pltpu.make_async_copy(
                    table_hbm.at[idx_v.at[pl.ds(j * gi, gi)]],
                    rows_v.at[slot, pl.ds(j * gi, gi)],
                    gsem,
                ).wait()

        def start_wb(g, slot):
            for k in range(_GB):
                pltpu.async_copy(
                    rows_v.at[slot, pl.ds(k * f_dim, f_dim)],
                    out_hbm.at[b0 + g * _GB + k],
                    osem,
                )

        def wait_wb(slot):
            for k in range(_GB):
                pltpu.make_async_copy(
                    rows_v.at[slot, pl.ds(k * f_dim, f_dim)],
                    out_hbm.at[b0],
                    osem,
                ).wait()

        fire(0, 0)
        fire(1, 1)
        wait_gathers(0)
        start_wb(0, 0)

        def body(i, carry):
            gb = 1 + 2 * i
            for b in range(2):
                g = gb + b
                slot = (1 + b) % 2
                other = 1 - slot
                wait_wb(other)
                fire(g + 1, other)
                wait_gathers(slot)
                start_wb(g, slot)
            return carry

        lax.fori_loop(0, (n_groups - 2) // 2, body, 0)

        g_last = n_groups - 1
        slot = g_last % 2
        wait_gathers(slot)
        start_wb(g_last, slot)
        wait_wb(1 - slot)
        wait_wb(slot)

    return emb_kernel(xt, table)


def kernel(x, embedding):
    b, f = x.shape
    v, d = embedding.shape
    xt = lax.bitcast_convert_type(x.astype(jnp.int32), jnp.float32).T
    return _gather_rows(xt, embedding, b, f, d)


# speedup vs baseline: 1.4845x; 1.0074x over previous
import functools

import jax
import jax.numpy as jnp
from jax import lax
from jax.experimental import pallas as pl
from jax.experimental.pallas import tpu as pltpu
from jax.experimental.pallas import tpu_sc as plsc

_NC = 2
_NS = 16
_IDXW = 128
_K = 4
_CG = _K * _IDXW


@functools.partial(jax.jit, static_argnames=("n_rows", "d"))
def _gather_rows(idx2d, table, n_rows, d):
    nw = _NC * _NS
    per_w = n_rows // nw
    n_groups = per_w // _CG
    idx_rows_per_w = per_w // _IDXW

    mesh = plsc.VectorSubcoreMesh(
        core_axis_name="c", subcore_axis_name="s",
        num_cores=_NC, num_subcores=_NS,
    )

    @functools.partial(
        pl.kernel,
        mesh=mesh,
        out_type=jax.ShapeDtypeStruct((n_rows, d), jnp.float32),
        scratch_types=[
            pltpu.VMEM((idx_rows_per_w, _IDXW), jnp.int32),
            pltpu.VMEM((2, _CG, d), jnp.float32),
            pltpu.SemaphoreType.DMA,
            pltpu.SemaphoreType.DMA,
        ],
        compiler_params=pltpu.CompilerParams(use_tc_tiling_on_sc=False),
    )
    def emb_kernel(idx_hbm, table_hbm, out_hbm, idx_v, rows_v, gsem, osem):
        wid = lax.axis_index("s") * _NC + lax.axis_index("c")
        idx_row_base = wid * idx_rows_per_w
        out_base = idx_row_base * _IDXW

        pltpu.sync_copy(idx_hbm.at[pl.ds(idx_row_base, idx_rows_per_w), :],
                        idx_v)

        def fire(g, slot):
            for j in range(_K):
                pltpu.async_copy(
                    table_hbm.at[idx_v.at[g * _K + j]],
                    rows_v.at[slot, pl.ds(j * _IDXW, _IDXW)],
                    gsem,
                )

        def wait_gathers(slot):
            for j in range(_K):
                pltpu.make_async_copy(
                    table_hbm.at[idx_v.at[j]],
                    rows_v.at[slot, pl.ds(j * _IDXW, _IDXW)],
                    gsem,
                ).wait()

        def start_wb(g, slot):
            pltpu.async_copy(
                rows_v.at[slot],
                out_hbm.at[pl.ds(out_base + g * _CG, _CG)],
                osem,
            )

        def wait_wb(slot):
            pltpu.make_async_copy(
                rows_v.at[slot],
                out_hbm.at[pl.ds(out_base, _CG)],
                osem,
            ).wait()

        fire(0, 0)
        fire(1, 1)
        wait_gathers(0)
        start_wb(0, 0)

        def body(i, carry):
            gb = 1 + 2 * i
            for b in range(2):
                g = gb + b
                slot = (1 + b) % 2
                other = 1 - slot
                wait_wb(other)
                fire(g + 1, other)
                wait_gathers(slot)
                start_wb(g, slot)
            return carry

        lax.fori_loop(0, (n_groups - 2) // 2, body, 0)

        g_last = n_groups - 1
        slot = g_last % 2
        wait_gathers(slot)
        start_wb(g_last, slot)
        wait_wb(1 - slot)
        wait_wb(slot)

    return emb_kernel(idx2d, table)


def kernel(x, embedding):
    b, f = x.shape
    v, d = embedding.shape
    n = b * f
    idx2d = x.reshape(n // _IDXW, _IDXW).astype(jnp.int32)
    out = _gather_rows(idx2d, embedding, n, d)
    return out.reshape(b, f, d)
